# Initial kernel scaffold; baseline (speedup 1.0000x reference)
import jax
import jax.numpy as jnp
from jax.experimental import pallas as pl


def kernel(counts, genes, gene_embedding):
    raise NotImplementedError("write your pallas kernel here")


# speedup vs baseline: 2.4862x; 2.4862x over previous
import functools

import jax
import jax.numpy as jnp
from jax import lax
from jax.experimental import pallas as pl
from jax.experimental.pallas import tpu as pltpu
from jax.experimental.pallas import tpu_sc as plsc

N_GENES = 100000
N_EMBED = 64
BATCH = 4096
SEQ = 200
N_FLAT = BATCH * SEQ

NW = 32
PER_W = N_FLAT // NW
CHUNK = 512
NCHUNK = PER_W // CHUNK
LANES = 16
VPR = N_EMBED // LANES


def _log1p_body(c_ref, o_ref):
    o_ref[...] = jnp.log1p(c_ref[...])


def _log1p_tc(counts):
    return pl.pallas_call(
        _log1p_body,
        out_shape=jax.ShapeDtypeStruct((BATCH, SEQ), jnp.float32),
        grid=(8,),
        in_specs=[pl.BlockSpec((BATCH // 8, SEQ), lambda i: (i, 0))],
        out_specs=pl.BlockSpec((BATCH // 8, SEQ), lambda i: (i, 0)),
    )(counts)


def _sc_body(table, genes, lp, out, idx_v, lp_v, rows_v, sem):
    cid = lax.axis_index("c")
    sid = lax.axis_index("s")
    wid = sid * 2 + cid
    base_w = wid * PER_W

    def chunk_body(k, carry):
        base = base_w + k * CHUNK
        pltpu.sync_copy(genes.at[pl.ds(base, CHUNK)], idx_v)
        pltpu.sync_copy(lp.at[pl.ds(base, CHUNK)], lp_v)
        pltpu.async_copy(table.at[idx_v], rows_v, sem).wait()

        def group_body(g, carry2):
            lp_vec = lp_v[pl.ds(g * LANES, LANES)]
            dnums = lax.GatherDimensionNumbers(
                offset_dims=(), collapsed_slice_dims=(0,), start_index_map=(0,)
            )
            for i in range(LANES):
                scale = lax.gather(
                    lp_vec,
                    jnp.full((LANES, 1), i, jnp.int32),
                    dnums,
                    slice_sizes=(1,),
                    mode=lax.GatherScatterMode.PROMISE_IN_BOUNDS,
                )
                r = g * LANES + i
                for j in range(VPR):
                    rows_v[r, pl.ds(j * LANES, LANES)] = (
                        rows_v[r, pl.ds(j * LANES, LANES)] * scale
                    )
            return carry2

        lax.fori_loop(0, CHUNK // LANES, group_body, 0)
        pltpu.sync_copy(rows_v, out.at[pl.ds(base, CHUNK)])
        return carry

    lax.fori_loop(0, NCHUNK, chunk_body, 0)


@functools.partial(jax.jit, static_argnums=())
def _run(counts, genes_flat, gene_embedding):
    lp = _log1p_tc(counts).reshape(N_FLAT)
    mesh = plsc.VectorSubcoreMesh(core_axis_name="c", subcore_axis_name="s")
    sc = pl.kernel(
        _sc_body,
        mesh=mesh,
        compiler_params=pltpu.CompilerParams(use_tc_tiling_on_sc=False),
        out_type=jax.ShapeDtypeStruct((N_FLAT, N_EMBED), jnp.float32),
        scratch_types=[
            pltpu.VMEM((CHUNK,), jnp.int32),
            pltpu.VMEM((CHUNK,), jnp.float32),
            pltpu.VMEM((CHUNK, N_EMBED), jnp.float32),
            pltpu.SemaphoreType.DMA,
        ],
    )
    out = sc(gene_embedding, genes_flat, lp)
    return out.reshape(BATCH, SEQ, N_EMBED)


def kernel(counts, genes, gene_embedding):
    genes_flat = genes.reshape(N_FLAT).astype(jnp.int32)
    return _run(counts, genes_flat, gene_embedding)


# speedup vs baseline: 2.8520x; 1.1471x over previous
import functools

import jax
import jax.numpy as jnp
from jax import lax
from jax.experimental import pallas as pl
from jax.experimental.pallas import tpu as pltpu
from jax.experimental.pallas import tpu_sc as plsc

N_GENES = 100000
N_EMBED = 64
BATCH = 4096
SEQ = 200
N_FLAT = BATCH * SEQ

NW = 32
PER_W = N_FLAT // NW
CHUNK = 512
NCH = PER_W // CHUNK
LANES = 16
VPR = N_EMBED // LANES


def _log1p_body(c_ref, o_ref):
    o_ref[...] = jnp.log1p(c_ref[...])


def _log1p_tc(counts):
    return pl.pallas_call(
        _log1p_body,
        out_shape=jax.ShapeDtypeStruct((BATCH, SEQ), jnp.float32),
        grid=(8,),
        in_specs=[pl.BlockSpec((BATCH // 8, SEQ), lambda i: (i, 0))],
        out_specs=pl.BlockSpec((BATCH // 8, SEQ), lambda i: (i, 0)),
    )(counts)


def _sc_body(table, genes2d, lp2d, out, idx_v, lp_v, rows0, rows1, g0, g1, so):
    cid = lax.axis_index("c")
    sid = lax.axis_index("s")
    wid = sid * 2 + cid
    row0 = wid * NCH
    base_w = wid * PER_W

    pltpu.sync_copy(genes2d.at[pl.ds(row0, NCH)], idx_v)
    pltpu.sync_copy(lp2d.at[pl.ds(row0, NCH)], lp_v)

    rows = (rows0, rows1)
    gsem = (g0, g1)

    pltpu.async_copy(table.at[idx_v.at[0]], rows0, g0)

    def scale_and_store(k, b):
        rb = rows[b]

        def group_body(g, carry):
            lp_vec = lp_v[k, pl.ds(g * LANES, LANES)]
            dnums = lax.GatherDimensionNumbers(
                offset_dims=(), collapsed_slice_dims=(0,), start_index_map=(0,)
            )
            for i in range(LANES):
                scale = lax.gather(
                    lp_vec,
                    jnp.full((LANES, 1), i, jnp.int32),
                    dnums,
                    slice_sizes=(1,),
                    mode=lax.GatherScatterMode.PROMISE_IN_BOUNDS,
                )
                r = g * LANES + i
                for j in range(VPR):
                    rb[r, pl.ds(j * LANES, LANES)] = (
                        rb[r, pl.ds(j * LANES, LANES)] * scale
                    )
            return carry

        lax.fori_loop(0, CHUNK // LANES, group_body, 0)
        pltpu.sync_copy(rb, out.at[pl.ds(base_w + k * CHUNK, CHUNK)])

    def pair_body(i, carry):
        k0 = i * 2
        for b in range(2):
            k = k0 + b
            nb = 1 - b

            @pl.when(k + 1 < NCH)
            def _():
                pltpu.async_copy(table.at[idx_v.at[k + 1]], rows[nb], gsem[nb])

            pltpu.make_async_copy(table.at[idx_v.at[k]], rows[b], gsem[b]).wait()
            scale_and_store(k, b)
        return carry

    lax.fori_loop(0, NCH // 2, pair_body, 0)


@functools.partial(jax.jit, static_argnums=())
def _run(counts, genes2d, gene_embedding):
    lp2d = _log1p_tc(counts).reshape(N_FLAT // CHUNK, CHUNK)
    mesh = plsc.VectorSubcoreMesh(core_axis_name="c", subcore_axis_name="s")
    sc = pl.kernel(
        _sc_body,
        mesh=mesh,
        compiler_params=pltpu.CompilerParams(use_tc_tiling_on_sc=False),
        out_type=jax.ShapeDtypeStruct((N_FLAT, N_EMBED), jnp.float32),
        scratch_types=[
            pltpu.VMEM((NCH, CHUNK), jnp.int32),
            pltpu.VMEM((NCH, CHUNK), jnp.float32),
            pltpu.VMEM((CHUNK, N_EMBED), jnp.float32),
            pltpu.VMEM((CHUNK, N_EMBED), jnp.float32),
            pltpu.SemaphoreType.DMA,
            pltpu.SemaphoreType.DMA,
            pltpu.SemaphoreType.DMA,
        ],
    )
    out = sc(gene_embedding, genes2d, lp2d)
    return out.reshape(BATCH, SEQ, N_EMBED)


def kernel(counts, genes, gene_embedding):
    genes2d = genes.reshape(N_FLAT // CHUNK, CHUNK).astype(jnp.int32)
    return _run(counts, genes2d, gene_embedding)


# speedup vs baseline: 2.9764x; 1.0436x over previous
import functools

import jax
import jax.numpy as jnp
from jax import lax
from jax.experimental import pallas as pl
from jax.experimental.pallas import tpu as pltpu
from jax.experimental.pallas import tpu_sc as plsc

N_GENES = 100000
N_EMBED = 64
BATCH = 4096
SEQ = 200

NW = 32
BPW = BATCH // NW
ROWS_PER_CHUNK = 2
NCH = BPW // ROWS_PER_CHUNK
LANES = 16
VPR = N_EMBED // LANES
NGRP = SEQ // LANES
TAIL0 = SEQ - LANES

_C = (
    0.04342890782205806,
    -0.40486717441854486,
    1.5939013634971635,
    -3.4924942798763934,
    5.0468760449737635,
    -2.7868129538668147,
)
_LN2 = 0.6931471805599453


def _log1p16(x):
    t = x + jnp.float32(1.0)
    ti = lax.bitcast_convert_type(t, jnp.int32)
    e = (ti >> 23) - 127
    mi = (ti & jnp.int32(0x007FFFFF)) | jnp.int32(0x3F800000)
    m = lax.bitcast_convert_type(mi, jnp.float32)
    p = jnp.full((LANES,), _C[0], jnp.float32)
    for c in _C[1:]:
        p = p * m + jnp.float32(c)
    return (e.astype(jnp.float32) + p) * jnp.float32(_LN2)


_DNUMS = lax.GatherDimensionNumbers(
    offset_dims=(), collapsed_slice_dims=(0,), start_index_map=(0,)
)


def _bcast(vec, i):
    return lax.gather(
        vec,
        jnp.full((LANES, 1), i, jnp.int32),
        _DNUMS,
        slice_sizes=(1,),
        mode=lax.GatherScatterMode.PROMISE_IN_BOUNDS,
    )


def _sc_body(table, genes, counts, out, idx_v, cnt_v, lp_s, r0, r1, r2, g0, g1, g2, o0, o1, o2):
    cid = lax.axis_index("c")
    sid = lax.axis_index("s")
    wid = sid * 2 + cid
    b0 = wid * BPW

    pltpu.sync_copy(genes.at[pl.ds(b0, BPW)], idx_v)
    pltpu.sync_copy(counts.at[pl.ds(b0, BPW)], cnt_v)

    rows = (r0, r1, r2)
    gsem = (g0, g1, g2)
    osem = (o0, o1, o2)

    def start_gather(k, b):
        for br in range(ROWS_PER_CHUNK):
            pltpu.async_copy(
                table.at[idx_v.at[k * ROWS_PER_CHUNK + br]],
                rows[b].at[br],
                gsem[b],
            )

    def wait_gather(b):
        for br in range(ROWS_PER_CHUNK):
            pltpu.make_async_copy(
                table.at[idx_v.at[0]], rows[b].at[br], gsem[b]
            ).wait()

    def start_out(k, b):
        pltpu.async_copy(
            rows[b], out.at[pl.ds(b0 + k * ROWS_PER_CHUNK, ROWS_PER_CHUNK)], osem[b]
        )

    def wait_out(b):
        pltpu.make_async_copy(
            rows[b], out.at[pl.ds(b0, ROWS_PER_CHUNK)], osem[b]
        ).wait()

    def scale(k, b):
        rb = rows[b]
        for br in range(ROWS_PER_CHUNK):
            lrow = k * ROWS_PER_CHUNK + br

            def lp_body(g, carry, br=br, lrow=lrow):
                lp_s[br, pl.ds(g * LANES, LANES)] = _log1p16(
                    cnt_v[lrow, pl.ds(g * LANES, LANES)]
                )
                return carry

            lax.fori_loop(0, NGRP, lp_body, 0)
            lp_s[br, pl.ds(TAIL0, LANES)] = _log1p16(cnt_v[lrow, pl.ds(TAIL0, LANES)])

        for br in range(ROWS_PER_CHUNK):

            def mul_body(g, carry, br=br):
                lp_vec = lp_s[br, pl.ds(g * LANES, LANES)]
                for i in range(LANES):
                    s = g * LANES + i
                    sc = _bcast(lp_vec, i)
                    for j in range(VPR):
                        rb[br, s, pl.ds(j * LANES, LANES)] = (
                            rb[br, s, pl.ds(j * LANES, LANES)] * sc
                        )
                return carry

            lax.fori_loop(0, NGRP, mul_body, 0)
            lp_vec = lp_s[br, pl.ds(TAIL0, LANES)]
            for i in range(LANES - (SEQ - NGRP * LANES), LANES):
                s = TAIL0 + i
                sc = _bcast(lp_vec, i)
                for j in range(VPR):
                    rb[br, s, pl.ds(j * LANES, LANES)] = (
                        rb[br, s, pl.ds(j * LANES, LANES)] * sc
                    )

    start_gather(0, 0)
    start_gather(1, 1)
    wait_gather(0)
    scale(0, 0)
    start_out(0, 0)
    start_gather(2, 2)

    def tri_body(i, carry):
        for j in range(3):
            k = 3 * i + 1 + j
            b = (1 + j) % 3
            wait_gather(b)
            scale(k, b)
            start_out(k, b)

            @pl.when(k + 2 < NCH)
            def _(b=b, k=k):
                nb = (b + 2) % 3
                wait_out(nb)
                start_gather(k + 2, nb)

        return carry

    lax.fori_loop(0, (NCH - 1) // 3, tri_body, 0)

    for kk in range(NCH - 3, NCH):
        wait_out(kk % 3)


def _run(counts, genes, gene_embedding):
    mesh = plsc.VectorSubcoreMesh(core_axis_name="c", subcore_axis_name="s")
    sc = pl.kernel(
        _sc_body,
        mesh=mesh,
        compiler_params=pltpu.CompilerParams(use_tc_tiling_on_sc=False),
        out_type=jax.ShapeDtypeStruct((BATCH, SEQ, N_EMBED), jnp.float32),
        scratch_types=[
            pltpu.VMEM((BPW, SEQ), jnp.int32),
            pltpu.VMEM((BPW, SEQ), jnp.float32),
            pltpu.VMEM((ROWS_PER_CHUNK, SEQ), jnp.float32),
            pltpu.VMEM((ROWS_PER_CHUNK, SEQ, N_EMBED), jnp.float32),
            pltpu.VMEM((ROWS_PER_CHUNK, SEQ, N_EMBED), jnp.float32),
            pltpu.VMEM((ROWS_PER_CHUNK, SEQ, N_EMBED), jnp.float32),
            pltpu.SemaphoreType.DMA,
            pltpu.SemaphoreType.DMA,
            pltpu.SemaphoreType.DMA,
            pltpu.SemaphoreType.DMA,
            pltpu.SemaphoreType.DMA,
            pltpu.SemaphoreType.DMA,
        ],
    )
    return sc(gene_embedding, genes, counts)


def kernel(counts, genes, gene_embedding):
    return _run(counts, genes.astype(jnp.int32), gene_embedding)


# speedup vs baseline: 3.0086x; 1.0108x over previous
import jax
import jax.numpy as jnp
from jax import lax
from jax.experimental import pallas as pl
from jax.experimental.pallas import tpu as pltpu
from jax.experimental.pallas import tpu_sc as plsc

N_GENES = 100000
N_EMBED = 64
BATCH = 4096
SEQ = 200

NW = 32
BPW = BATCH // NW
NCH = BPW
NBUF = 6
LOOKAHEAD = 3
LANES = 16
VPR = N_EMBED // LANES
NGRP = SEQ // LANES
TAIL0 = SEQ - LANES
TAILI = LANES - (SEQ - NGRP * LANES)

_C = (
    0.04342890782205806,
    -0.40486717441854486,
    1.5939013634971635,
    -3.4924942798763934,
    5.0468760449737635,
    -2.7868129538668147,
)
_LN2 = 0.6931471805599453


def _log1p16(x):
    t = x + jnp.float32(1.0)
    ti = lax.bitcast_convert_type(t, jnp.int32)
    e = (ti >> 23) - 127
    mi = (ti & jnp.int32(0x007FFFFF)) | jnp.int32(0x3F800000)
    m = lax.bitcast_convert_type(mi, jnp.float32)
    p = jnp.full((LANES,), _C[0], jnp.float32)
    for c in _C[1:]:
        p = p * m + jnp.float32(c)
    return (e.astype(jnp.float32) + p) * jnp.float32(_LN2)


_DNUMS = lax.GatherDimensionNumbers(
    offset_dims=(), collapsed_slice_dims=(0,), start_index_map=(0,)
)


def _bcast(vec, i):
    return lax.gather(
        vec,
        jnp.full((LANES, 1), i, jnp.int32),
        _DNUMS,
        slice_sizes=(1,),
        mode=lax.GatherScatterMode.PROMISE_IN_BOUNDS,
    )


def _sc_body(table, genes, counts, out, idx_v, lp_v, bufs, gsem, osem):
    cid = lax.axis_index("c")
    sid = lax.axis_index("s")
    wid = sid * 2 + cid
    b0 = wid * BPW

    pltpu.sync_copy(genes.at[pl.ds(b0, BPW)], idx_v)
    pltpu.sync_copy(counts.at[pl.ds(b0, BPW)], lp_v)

    def start_gather(k, b):
        pltpu.async_copy(table.at[idx_v.at[k]], bufs[b].at[0], gsem[b])

    def wait_gather(b):
        pltpu.make_async_copy(table.at[idx_v.at[0]], bufs[b].at[0], gsem[b]).wait()

    def start_out(k, b):
        pltpu.async_copy(bufs[b], out.at[pl.ds(b0 + k, 1)], osem[b])

    def wait_out(b):
        pltpu.make_async_copy(bufs[b], out.at[pl.ds(b0, 1)], osem[b]).wait()

    for k in range(LOOKAHEAD):
        start_gather(k, k)

    def lp_row(r, carry):
        xt = lp_v[r, pl.ds(TAIL0, LANES)]

        def lp_g(g, c):
            lp_v[r, pl.ds(g * LANES, LANES)] = _log1p16(lp_v[r, pl.ds(g * LANES, LANES)])
            return c

        lax.fori_loop(0, NGRP, lp_g, 0)
        lp_v[r, pl.ds(TAIL0, LANES)] = _log1p16(xt)
        return carry

    lax.fori_loop(0, BPW, lp_row, 0)

    def scale(k, b):
        rb = bufs[b]

        def mul_body(g, carry):
            lp_vec = lp_v[k, pl.ds(g * LANES, LANES)]
            for i in range(LANES):
                s = g * LANES + i
                sc = _bcast(lp_vec, i)
                for j in range(VPR):
                    rb[0, s, pl.ds(j * LANES, LANES)] = (
                        rb[0, s, pl.ds(j * LANES, LANES)] * sc
                    )
            return carry

        lax.fori_loop(0, NGRP, mul_body, 0)
        lp_vec = lp_v[k, pl.ds(TAIL0, LANES)]
        for i in range(TAILI, LANES):
            s = TAIL0 + i
            sc = _bcast(lp_vec, i)
            for j in range(VPR):
                rb[0, s, pl.ds(j * LANES, LANES)] = (
                    rb[0, s, pl.ds(j * LANES, LANES)] * sc
                )

    def step(k, b):
        wait_gather(b)
        scale(k, b)
        start_out(k, b)

        @pl.when(k + LOOKAHEAD < NCH)
        def _():
            nb = (b + LOOKAHEAD) % NBUF

            @pl.when(k >= NBUF - LOOKAHEAD)
            def _():
                wait_out(nb)

            start_gather(k + LOOKAHEAD, nb)

    def six_body(i, carry):
        for j in range(NBUF):
            step(i * NBUF + j, j)
        return carry

    lax.fori_loop(0, NCH // NBUF, six_body, 0)
    for k in range(NCH - NCH % NBUF, NCH):
        step(k, k % NBUF)
    for k in range(NCH - NBUF, NCH):
        wait_out(k % NBUF)


def _run(counts, genes, gene_embedding):
    mesh = plsc.VectorSubcoreMesh(core_axis_name="c", subcore_axis_name="s")

    def body(table, genes_, counts_, out, idx_v, lp_v, b0, b1, b2, b3, b4, b5,
             g0, g1, g2, g3, g4, g5, o0, o1, o2, o3, o4, o5):
        _sc_body(
            table, genes_, counts_, out, idx_v, lp_v,
            (b0, b1, b2, b3, b4, b5),
            (g0, g1, g2, g3, g4, g5),
            (o0, o1, o2, o3, o4, o5),
        )

    sc = pl.kernel(
        body,
        mesh=mesh,
        compiler_params=pltpu.CompilerParams(use_tc_tiling_on_sc=False),
        out_type=jax.ShapeDtypeStruct((BATCH, SEQ, N_EMBED), jnp.float32),
        scratch_types=[
            pltpu.VMEM((BPW, SEQ), jnp.int32),
            pltpu.VMEM((BPW, SEQ), jnp.float32),
        ]
        + [pltpu.VMEM((1, SEQ, N_EMBED), jnp.float32)] * NBUF
        + [pltpu.SemaphoreType.DMA] * (2 * NBUF),
    )
    return sc(gene_embedding, genes, counts)


def kernel(counts, genes, gene_embedding):
    return _run(counts, genes.astype(jnp.int32), gene_embedding)


# speedup vs baseline: 4.1997x; 1.3959x over previous
import jax
import jax.numpy as jnp
from jax import lax
from jax.experimental import pallas as pl
from jax.experimental.pallas import tpu as pltpu
from jax.experimental.pallas import tpu_sc as plsc

N_GENES = 100000
N_EMBED = 64
BATCH = 4096
SEQ = 200

NW = 32
BPW = BATCH // NW
NCH = BPW
NBUF = 6
LOOKAHEAD = 3
LANES = 16
VPR = N_EMBED // LANES
NGRP = SEQ // LANES
TAIL0 = SEQ - LANES
TAILI = LANES - (SEQ - NGRP * LANES)

_C = (
    0.04342890782205806,
    -0.40486717441854486,
    1.5939013634971635,
    -3.4924942798763934,
    5.0468760449737635,
    -2.7868129538668147,
)
_LN2 = 0.6931471805599453


def _log1p16(x):
    t = x + jnp.float32(1.0)
    ti = lax.bitcast_convert_type(t, jnp.int32)
    e = (ti >> 23) - 127
    mi = (ti & jnp.int32(0x007FFFFF)) | jnp.int32(0x3F800000)
    m = lax.bitcast_convert_type(mi, jnp.float32)
    p = jnp.full((LANES,), _C[0], jnp.float32)
    for c in _C[1:]:
        p = p * m + jnp.float32(c)
    return (e.astype(jnp.float32) + p) * jnp.float32(_LN2)


_DNUMS = lax.GatherDimensionNumbers(
    offset_dims=(), collapsed_slice_dims=(0,), start_index_map=(0,)
)


def _bcast(vec, i):
    return lax.gather(
        vec,
        jnp.full((LANES, 1), i, jnp.int32),
        _DNUMS,
        slice_sizes=(1,),
        mode=lax.GatherScatterMode.PROMISE_IN_BOUNDS,
    )


def _sc_body(table, genes, counts, out, idx_v, cnt_v, bufs, gsem, osem):
    cid = lax.axis_index("c")
    sid = lax.axis_index("s")
    wid = sid * 2 + cid
    b0 = wid * BPW

    pltpu.sync_copy(genes.at[pl.ds(b0, BPW)], idx_v)
    pltpu.sync_copy(counts.at[pl.ds(b0, BPW)], cnt_v)

    def start_gather(k, b):
        pltpu.async_copy(table.at[idx_v.at[k]], bufs[b].at[0], gsem[b])

    def wait_gather(b):
        pltpu.make_async_copy(table.at[idx_v.at[0]], bufs[b].at[0], gsem[b]).wait()

    def start_out(k, b):
        pltpu.async_copy(bufs[b], out.at[pl.ds(b0 + k, 1)], osem[b])

    def wait_out(b):
        pltpu.make_async_copy(bufs[b], out.at[pl.ds(b0, 1)], osem[b]).wait()

    def scale(k, b):
        rb = bufs[b]

        @plsc.parallel_loop(0, NGRP)
        def _(g):
            lp_vec = _log1p16(cnt_v[k, pl.ds(g * LANES, LANES)])
            for i in range(LANES):
                s = g * LANES + i
                sc = _bcast(lp_vec, i)
                for j in range(VPR):
                    rb[0, s, pl.ds(j * LANES, LANES)] = (
                        rb[0, s, pl.ds(j * LANES, LANES)] * sc
                    )

        lp_vec = _log1p16(cnt_v[k, pl.ds(TAIL0, LANES)])
        for i in range(TAILI, LANES):
            s = TAIL0 + i
            sc = _bcast(lp_vec, i)
            for j in range(VPR):
                rb[0, s, pl.ds(j * LANES, LANES)] = (
                    rb[0, s, pl.ds(j * LANES, LANES)] * sc
                )

    def step(k, b):
        wait_gather(b)
        scale(k, b)
        start_out(k, b)

        @pl.when(k + LOOKAHEAD < NCH)
        def _():
            nb = (b + LOOKAHEAD) % NBUF

            @pl.when(k >= NBUF - LOOKAHEAD)
            def _():
                wait_out(nb)

            start_gather(k + LOOKAHEAD, nb)

    for k in range(LOOKAHEAD):
        start_gather(k, k)

    def six_body(i, carry):
        for j in range(NBUF):
            step(i * NBUF + j, j)
        return carry

    lax.fori_loop(0, NCH // NBUF, six_body, 0)
    for k in range(NCH - NCH % NBUF, NCH):
        step(k, k % NBUF)
    for k in range(NCH - NBUF, NCH):
        wait_out(k % NBUF)


def _run(counts, genes, gene_embedding):
    mesh = plsc.VectorSubcoreMesh(core_axis_name="c", subcore_axis_name="s")

    def body(table, genes_, counts_, out, idx_v, cnt_v, b0, b1, b2, b3, b4, b5,
             g0, g1, g2, g3, g4, g5, o0, o1, o2, o3, o4, o5):
        _sc_body(
            table, genes_, counts_, out, idx_v, cnt_v,
            (b0, b1, b2, b3, b4, b5),
            (g0, g1, g2, g3, g4, g5),
            (o0, o1, o2, o3, o4, o5),
        )

    sc = pl.kernel(
        body,
        mesh=mesh,
        compiler_params=pltpu.CompilerParams(use_tc_tiling_on_sc=False),
        out_type=jax.ShapeDtypeStruct((BATCH, SEQ, N_EMBED), jnp.float32),
        scratch_types=[
            pltpu.VMEM((BPW, SEQ), jnp.int32),
            pltpu.VMEM((BPW, SEQ), jnp.float32),
        ]
        + [pltpu.VMEM((1, SEQ, N_EMBED), jnp.float32)] * NBUF
        + [pltpu.SemaphoreType.DMA] * (2 * NBUF),
    )
    return sc(gene_embedding, genes, counts)


def kernel(counts, genes, gene_embedding):
    return _run(counts, genes.astype(jnp.int32), gene_embedding)


# speedup vs baseline: 4.2136x; 1.0033x over previous
import jax
import jax.numpy as jnp
from jax import lax
from jax.experimental import pallas as pl
from jax.experimental.pallas import tpu as pltpu
from jax.experimental.pallas import tpu_sc as plsc

N_GENES = 100000
N_EMBED = 64
BATCH = 4096
SEQ = 200

NW = 32
BPW = BATCH // NW
NCH = BPW
NBUF = 3
LANES = 16
VPR = N_EMBED // LANES
NGRP = SEQ // LANES
TAIL0 = SEQ - LANES
TAILI = LANES - (SEQ - NGRP * LANES)
PROWS = SEQ // 2

_C = (
    0.04342890782205806,
    -0.40486717441854486,
    1.5939013634971635,
    -3.4924942798763934,
    5.0468760449737635,
    -2.7868129538668147,
)
_LN2 = 0.6931471805599453


def _log1p16(x):
    t = x + jnp.float32(1.0)
    ti = lax.bitcast_convert_type(t, jnp.int32)
    e = (ti >> 23) - 127
    mi = (ti & jnp.int32(0x007FFFFF)) | jnp.int32(0x3F800000)
    m = lax.bitcast_convert_type(mi, jnp.float32)
    p = jnp.full((LANES,), _C[0], jnp.float32)
    for c in _C[1:]:
        p = p * m + jnp.float32(c)
    return (e.astype(jnp.float32) + p) * jnp.float32(_LN2)


_DNUMS = lax.GatherDimensionNumbers(
    offset_dims=(), collapsed_slice_dims=(0,), start_index_map=(0,)
)


def _bcast(vec, i):
    return lax.gather(
        vec,
        jnp.full((LANES, 1), i, jnp.int32),
        _DNUMS,
        slice_sizes=(1,),
        mode=lax.GatherScatterMode.PROMISE_IN_BOUNDS,
    )


def _sc_body(table, genes, counts, out, idx_v, cnt_v, gbufs, pbufs, gsem, osem):
    cid = lax.axis_index("c")
    sid = lax.axis_index("s")
    wid = sid * 2 + cid
    b0 = wid * BPW

    pltpu.sync_copy(genes.at[pl.ds(b0, BPW)], idx_v)
    pltpu.sync_copy(counts.at[pl.ds(b0, BPW)], cnt_v)

    def start_gather(k, b):
        pltpu.async_copy(table.at[idx_v.at[k]], gbufs[b], gsem[b])

    def wait_gather(b):
        pltpu.make_async_copy(table.at[idx_v.at[0]], gbufs[b], gsem[b]).wait()

    def start_out(k, b):
        pltpu.async_copy(pbufs[b], out.at[pl.ds((b0 + k) * PROWS, PROWS)], osem[b])

    def wait_out(b):
        pltpu.make_async_copy(pbufs[b], out.at[pl.ds(0, PROWS)], osem[b]).wait()

    def scale(k, b):
        gb = gbufs[b]
        pb = pbufs[b]

        @plsc.parallel_loop(0, NGRP)
        def _(g):
            lp_vec = _log1p16(cnt_v[k, pl.ds(g * LANES, LANES)])
            for i in range(LANES):
                s = g * LANES + i
                prow = g * (LANES // 2) + i // 2
                pcol = (i % 2) * N_EMBED
                sc = _bcast(lp_vec, i)
                for j in range(VPR):
                    pb[prow, pl.ds(pcol + j * LANES, LANES)] = (
                        gb[s, pl.ds(j * LANES, LANES)] * sc
                    )

        lp_vec = _log1p16(cnt_v[k, pl.ds(TAIL0, LANES)])
        for i in range(TAILI, LANES):
            s = TAIL0 + i
            prow = s // 2
            pcol = (s % 2) * N_EMBED
            sc = _bcast(lp_vec, i)
            for j in range(VPR):
                pb[prow, pl.ds(pcol + j * LANES, LANES)] = (
                    gb[s, pl.ds(j * LANES, LANES)] * sc
                )

    def step(k, b):
        wait_gather(b)

        @pl.when(k >= NBUF)
        def _():
            wait_out(b)

        scale(k, b)
        start_out(k, b)

        @pl.when(k + NBUF < NCH)
        def _():
            start_gather(k + NBUF, b)

    for k in range(NBUF):
        start_gather(k, k)

    def tri_body(i, carry):
        for j in range(NBUF):
            step(i * NBUF + j, j)
        return carry

    lax.fori_loop(0, NCH // NBUF, tri_body, 0)
    for k in range(NCH - NCH % NBUF, NCH):
        step(k, k % NBUF)
    for k in range(NCH - NBUF, NCH):
        wait_out(k % NBUF)


def _run(counts, genes, gene_embedding):
    mesh = plsc.VectorSubcoreMesh(core_axis_name="c", subcore_axis_name="s")

    def body(table, genes_, counts_, out, idx_v, cnt_v, gb0, gb1, gb2,
             pb0, pb1, pb2, g0, g1, g2, o0, o1, o2):
        _sc_body(
            table, genes_, counts_, out, idx_v, cnt_v,
            (gb0, gb1, gb2), (pb0, pb1, pb2),
            (g0, g1, g2), (o0, o1, o2),
        )

    sc = pl.kernel(
        body,
        mesh=mesh,
        compiler_params=pltpu.CompilerParams(use_tc_tiling_on_sc=False),
        out_type=jax.ShapeDtypeStruct((BATCH * SEQ // 2, 2 * N_EMBED), jnp.float32),
        scratch_types=[
            pltpu.VMEM((BPW, SEQ), jnp.int32),
            pltpu.VMEM((BPW, SEQ), jnp.float32),
        ]
        + [pltpu.VMEM((SEQ, N_EMBED), jnp.float32)] * NBUF
        + [pltpu.VMEM((PROWS, 2 * N_EMBED), jnp.float32)] * NBUF
        + [pltpu.SemaphoreType.DMA] * (2 * NBUF),
    )
    packed = sc(gene_embedding, genes, counts)
    return packed.reshape(BATCH, SEQ, N_EMBED)


def kernel(counts, genes, gene_embedding):
    return _run(counts, genes.astype(jnp.int32), gene_embedding)
